# baseline (speedup 1.0000x reference)
import functools

import jax
import jax.numpy as jnp
from jax import lax
from jax.experimental import pallas as pl
from jax.experimental.pallas import tpu as pltpu
from jax.experimental.pallas import tpu_sc as plsc

N = 10000
E = 320000
D = 128
C_OUT = 40

NC = 2
NS = 16
NW = NC * NS
EPW = E // NW
CH = 80
NCH = EPW // CH
RPT = 624
ZR = 48
TAIL = N - NS * RPT
DEGW = 128

BLK = 1000
NBLK = N // BLK

_sc_mesh = plsc.VectorSubcoreMesh(core_axis_name="c", subcore_axis_name="s")


NSLOT = 3


@functools.partial(
    pl.kernel,
    mesh=_sc_mesh,
    out_type=jax.ShapeDtypeStruct((NC, N, D), jnp.float32),
    scratch_types=[
        pltpu.VMEM((NSLOT, CH), jnp.int32),
        pltpu.VMEM((CH,), jnp.int32),
        pltpu.VMEM((CH,), jnp.int32),
        pltpu.VMEM((CH,), jnp.int32),
        pltpu.VMEM((ZR, D), jnp.float32),
        pltpu.VMEM((NSLOT, CH, D), jnp.float32),
        pltpu.VMEM_SHARED((N, D), jnp.float32),
        pltpu.SemaphoreType.DMA((NSLOT,)),
        pltpu.SemaphoreType.DMA((NSLOT,)),
        pltpu.SemaphoreType.DMA((NSLOT,)),
        pltpu.SemaphoreType.DMA((NSLOT,)),
    ],
)
def _sc_aggregate(src_hbm, dst_hbm, hs_hbm, zeros_hbm, out_hbm,
                  sidx_v, dst0_v, dst1_v, dst2_v, zero_v, rows_v, acc_sh,
                  gsems, dsems, ssems, isems):
    c = lax.axis_index("c")
    s = lax.axis_index("s")
    wid = c * NS + s
    pltpu.sync_copy(zeros_hbm, zero_v)
    for k in range(RPT // ZR):
        pltpu.sync_copy(zero_v, acc_sh.at[pl.ds(s * RPT + k * ZR, ZR)])

    @pl.when(s == 0)
    def _():
        pltpu.sync_copy(zero_v.at[pl.ds(0, TAIL)],
                        acc_sh.at[pl.ds(NS * RPT, TAIL)])

    dbufs = (dst0_v, dst1_v, dst2_v)
    for j in range(NSLOT):
        pltpu.sync_copy(src_hbm.at[wid, j], sidx_v.at[j])
    plsc.subcore_barrier()

    for j in range(2):
        pltpu.async_copy(dst_hbm.at[wid, j], dbufs[j], dsems.at[j])
        pltpu.async_copy(hs_hbm.at[sidx_v.at[j]], rows_v.at[j], gsems.at[j])

    def scat_wait(j):
        pltpu.make_async_copy(rows_v.at[j], acc_sh.at[dbufs[j]],
                              ssems.at[j]).wait()

    def visit(ci, j, wait_prev_scatter=True):
        jr = (j + 2) % NSLOT
        pltpu.make_async_copy(hs_hbm.at[sidx_v.at[j]], rows_v.at[j],
                              gsems.at[j]).wait()

        @pl.when(ci + NSLOT < NCH)
        def _():
            pltpu.async_copy(src_hbm.at[wid, ci + NSLOT], sidx_v.at[j],
                             isems.at[j])

        pltpu.make_async_copy(dst_hbm.at[wid, 0], dbufs[j],
                              dsems.at[j]).wait()
        pltpu.async_copy(rows_v.at[j], acc_sh.at[dbufs[j]], ssems.at[j],
                         add=True)

        @pl.when(ci + 2 < NCH)
        def _():
            if wait_prev_scatter:
                scat_wait(jr)
                pltpu.make_async_copy(src_hbm.at[wid, 0], sidx_v.at[jr],
                                      isems.at[jr]).wait()
            pltpu.async_copy(hs_hbm.at[sidx_v.at[jr]], rows_v.at[jr],
                             gsems.at[jr])
            pltpu.async_copy(dst_hbm.at[wid, ci + 2], dbufs[jr],
                             dsems.at[jr])

    visit(0, 0, wait_prev_scatter=False)
    visit(1, 1)
    visit(2, 2)

    def body(g, carry):
        visit(3 * g, 0)
        visit(3 * g + 1, 1)
        visit(3 * g + 2, 2)
        return carry

    lax.fori_loop(1, (NCH - 3) // 3 + 1, body, 0)
    visit(NCH - 2, (NCH - 2) % NSLOT)
    visit(NCH - 1, (NCH - 1) % NSLOT)
    for j in range(NSLOT):
        scat_wait(j)
    plsc.subcore_barrier()
    pltpu.sync_copy(acc_sh.at[pl.ds(s * RPT, RPT)],
                    out_hbm.at[c, pl.ds(s * RPT, RPT)])

    @pl.when(s == 0)
    def _():
        pltpu.sync_copy(acc_sh.at[pl.ds(NS * RPT, TAIL)],
                        out_hbm.at[c, pl.ds(NS * RPT, TAIL)])


def _row_spec(w):
    return pl.BlockSpec((BLK, w), lambda i: (i, 0))


def _row_spec2(w):
    return pl.BlockSpec((BLK, w), lambda i: (i % NBLK, 0))


def _pair_spec():
    return pl.BlockSpec((2, BLK, D), lambda i: (0, i, 0))


def _pair_spec2():
    return pl.BlockSpec((2, BLK, D), lambda i: (0, i % NBLK, 0))


def _full_spec(shape):
    nd = len(shape)
    return pl.BlockSpec(shape, lambda i: (0,) * nd)


def _dinv(dp):
    deg = 1.0 + dp[0, :, 0:1] + dp[1, :, 0:1]
    return lax.rsqrt(deg)


def _kan_body(x_ref, w1t_ref, b1_ref, w2t_ref, b2_ref, h_ref):
    h = jnp.dot(x_ref[...], w1t_ref[...], preferred_element_type=jnp.float32)
    h = jax.nn.gelu(h + b1_ref[...])
    h = jnp.dot(h, w2t_ref[...], preferred_element_type=jnp.float32) + b2_ref[...]
    h_ref[...] = h


def _tc_kan(x, w1t, b1, w2t, b2):
    return pl.pallas_call(
        _kan_body,
        grid=(NBLK,),
        in_specs=[_row_spec(D),
                  _full_spec((D, D)), _full_spec((1, D)),
                  _full_spec((D, D)), _full_spec((1, D))],
        out_specs=_row_spec(D),
        out_shape=jax.ShapeDtypeStruct((N, D), jnp.float32),
    )(x, w1t, b1, w2t, b2)


def _scale_body(h_ref, dp_ref, hs_ref):
    hs_ref[...] = h_ref[...] * _dinv(dp_ref[...])


def _tc_scale(h, dp):
    return pl.pallas_call(
        _scale_body,
        grid=(NBLK,),
        in_specs=[_row_spec(D), _pair_spec()],
        out_specs=_row_spec(D),
        out_shape=jax.ShapeDtypeStruct((N, D), jnp.float32),
    )(h, dp)


def _accum_stats(v, st_s):
    s1 = jnp.sum(v, axis=0, keepdims=True)
    s2 = jnp.sum(v * v, axis=0, keepdims=True)
    st = jnp.concatenate([s1, s2, jnp.zeros((6, D), v.dtype)], axis=0)

    @pl.when(pl.program_id(0) == 0)
    def _():
        st_s[...] = st

    @pl.when(pl.program_id(0) > 0)
    def _():
        st_s[...] = st_s[...] + st


def _bn(v, st, gamma, beta):
    mu = st[0:1, :] * (1.0 / N)
    var = st[1:2, :] * (1.0 / N) - mu * mu
    return gamma * (v - mu) * lax.rsqrt(var + 1e-5) + beta


def _comb_phase1(ap_ref, hs_ref, dp_ref, b_ref, v_s, st_s):
    dinv = _dinv(dp_ref[...])
    v = dinv * (ap_ref[0] + ap_ref[1] + hs_ref[...]) + b_ref[...]
    i = pl.program_id(0)
    v_s[pl.ds(i * BLK, BLK), :] = v
    _accum_stats(v, st_s)


def _bnkan_body(ap_ref, hs_ref, dp_ref, b_ref, g_ref, be_ref,
                w1t_ref, b1_ref, w2t_ref, b2_ref, h_ref, hs1_ref, v_s, st_s):
    i = pl.program_id(0)

    @pl.when(i < NBLK)
    def _():
        _comb_phase1(ap_ref, hs_ref, dp_ref, b_ref, v_s, st_s)

    @pl.when(i >= NBLK)
    def _():
        dinv = _dinv(dp_ref[...])
        v = v_s[pl.ds((i - NBLK) * BLK, BLK), :]
        h = _bn(v, st_s[...], g_ref[...], be_ref[...])
        h_ref[...] = h
        t = jnp.dot(h, w1t_ref[...], preferred_element_type=jnp.float32)
        t = jax.nn.gelu(t + b1_ref[...])
        t = jnp.dot(t, w2t_ref[...],
                    preferred_element_type=jnp.float32) + b2_ref[...]
        hs1_ref[...] = t * dinv


def _tc_comb_bn_kan(ap, hs, dp, bias, gamma, beta, w1t, b1, w2t, b2):
    return pl.pallas_call(
        _bnkan_body,
        grid=(2 * NBLK,),
        in_specs=[_pair_spec2(), _row_spec2(D), _pair_spec2(),
                  _full_spec((1, D)),
                  _full_spec((1, D)), _full_spec((1, D)),
                  _full_spec((D, D)), _full_spec((1, D)),
                  _full_spec((D, D)), _full_spec((1, D))],
        out_specs=[_row_spec2(D), _row_spec2(D)],
        out_shape=[jax.ShapeDtypeStruct((N, D), jnp.float32),
                   jax.ShapeDtypeStruct((N, D), jnp.float32)],
        scratch_shapes=[pltpu.VMEM((N, D), jnp.float32),
                        pltpu.VMEM((8, D), jnp.float32)],
    )(ap, hs, dp, bias, gamma, beta, w1t, b1, w2t, b2)


_G = [i * 0.5 - 2.5 for i in range(11)]


def _spline_head(z, bwt, swt, nseg):
    out = jnp.dot(jax.nn.silu(z), bwt, preferred_element_type=jnp.float32)
    bas = [((z >= _G[k]) & (z < _G[k + 1])).astype(z.dtype)
           for k in range(10)]
    for p in range(1, 4):
        bas = [(z - _G[k]) / (_G[k + p] - _G[k]) * bas[k]
               + (_G[k + p + 1] - z) / (_G[k + p + 1] - _G[k + 1])
               * bas[k + 1]
               for k in range(10 - p)]
    for j in range(7):
        out = out + jnp.dot(bas[j], swt[j],
                            preferred_element_type=jnp.float32)
    return out


def _head_pre_body(x_ref, h1_ref, bwt_ref, swt_ref, out_ref):
    z = jnp.concatenate([x_ref[...], h1_ref[...]], axis=1)
    out_ref[...] = _spline_head(z, bwt_ref[...], swt_ref[...], 2)


def _tc_head_pre(x, h1, bwt_xh, swt_xh):
    return pl.pallas_call(
        _head_pre_body,
        grid=(NBLK,),
        in_specs=[_row_spec(D), _row_spec(D),
                  _full_spec((2 * D, C_OUT)), _full_spec((7, 2 * D, C_OUT))],
        out_specs=_row_spec(C_OUT),
        out_shape=jax.ShapeDtypeStruct((N, C_OUT), jnp.float32),
    )(x, h1, bwt_xh, swt_xh)


def _head_body(ap_ref, hs_ref, dp_ref, b_ref, g_ref, be_ref,
               pre_ref, bwt_ref, swt_ref, out_ref, v_s, st_s):
    i = pl.program_id(0)

    @pl.when(i < NBLK)
    def _():
        _comb_phase1(ap_ref, hs_ref, dp_ref, b_ref, v_s, st_s)

    @pl.when(i >= NBLK)
    def _():
        v2 = v_s[pl.ds((i - NBLK) * BLK, BLK), :]
        h2 = _bn(v2, st_s[...], g_ref[...], be_ref[...])
        out_ref[...] = pre_ref[...] + _spline_head(h2, bwt_ref[...],
                                                   swt_ref[...], 1)


def _tc_comb_head(ap, hs, dp, bias, gamma, beta, pre, bwt_h2, swt_h2):
    return pl.pallas_call(
        _head_body,
        grid=(2 * NBLK,),
        in_specs=[_pair_spec2(), _row_spec2(D), _pair_spec2(),
                  _full_spec((1, D)),
                  _full_spec((1, D)), _full_spec((1, D)),
                  _row_spec2(C_OUT),
                  _full_spec((D, C_OUT)), _full_spec((7, D, C_OUT))],
        out_specs=_row_spec2(C_OUT),
        out_shape=jax.ShapeDtypeStruct((N, C_OUT), jnp.float32),
        scratch_shapes=[pltpu.VMEM((N, D), jnp.float32),
                        pltpu.VMEM((8, D), jnp.float32)],
    )(ap, hs, dp, bias, gamma, beta, pre, bwt_h2, swt_h2)


def kernel(x, edge_index, fc1_w0, fc1_b0, fc2_w0, fc2_b0, conv_b0, gamma0,
           beta0, fc1_w1, fc1_b1, fc2_w1, fc2_b1, conv_b1, gamma1, beta1,
           base_w, spline_w):
    src = edge_index[0].reshape(NW, NCH, CH)
    dst = edge_index[1].reshape(NW, NCH, CH)
    zeros_d = jnp.zeros((ZR, D), jnp.float32)
    ones_nd = jnp.ones((N, D), jnp.float32)
    degp = _sc_aggregate(jnp.bitwise_and(src, 4095), dst, ones_nd, zeros_d)

    h0 = _tc_kan(x, fc1_w0.T, fc1_b0[None], fc2_w0.T, fc2_b0[None])
    hs0 = _tc_scale(h0, degp)
    acc0 = _sc_aggregate(src, dst, hs0, zeros_d)
    h1, hs1 = _tc_comb_bn_kan(acc0, hs0, degp, conv_b0[None],
                              gamma0[None], beta0[None], fc1_w1.T,
                              fc1_b1[None], fc2_w1.T, fc2_b1[None])

    swt = jnp.transpose(spline_w, (2, 1, 0))
    bwt = base_w.T
    acc1 = _sc_aggregate(src, dst, hs1, zeros_d)
    pre = _tc_head_pre(x, h1, bwt[:2 * D], swt[:, :2 * D, :])
    return _tc_comb_head(acc1, hs1, degp, conv_b1[None],
                         gamma1[None], beta1[None], pre,
                         bwt[2 * D:], swt[:, 2 * D:, :])


# speedup vs baseline: 1.0022x; 1.0022x over previous
import functools

import jax
import jax.numpy as jnp
from jax import lax
from jax.experimental import pallas as pl
from jax.experimental.pallas import tpu as pltpu
from jax.experimental.pallas import tpu_sc as plsc

N = 10000
E = 320000
D = 128
C_OUT = 40

NC = 2
NS = 16
NW = NC * NS
EPW = E // NW
CH = 80
NCH = EPW // CH
RPT = 624
ZR = 48
TAIL = N - NS * RPT
DEGW = 128

BLK = 1000
NBLK = N // BLK

_sc_mesh = plsc.VectorSubcoreMesh(core_axis_name="c", subcore_axis_name="s")


NSLOT = 3


@functools.partial(
    pl.kernel,
    mesh=_sc_mesh,
    out_type=jax.ShapeDtypeStruct((NC, N, D), jnp.float32),
    scratch_types=[
        pltpu.VMEM((NSLOT, CH), jnp.int32),
        pltpu.VMEM((CH,), jnp.int32),
        pltpu.VMEM((CH,), jnp.int32),
        pltpu.VMEM((CH,), jnp.int32),
        pltpu.VMEM((ZR, D), jnp.float32),
        pltpu.VMEM((NSLOT, CH, D), jnp.float32),
        pltpu.VMEM_SHARED((N, D), jnp.float32),
        pltpu.SemaphoreType.DMA((NSLOT,)),
        pltpu.SemaphoreType.DMA((NSLOT,)),
        pltpu.SemaphoreType.DMA((NSLOT,)),
        pltpu.SemaphoreType.DMA((NSLOT,)),
    ],
)
def _sc_aggregate(src_hbm, dst_hbm, hs_hbm, zeros_hbm, out_hbm,
                  sidx_v, dst0_v, dst1_v, dst2_v, zero_v, rows_v, acc_sh,
                  gsems, dsems, ssems, isems):
    c = lax.axis_index("c")
    s = lax.axis_index("s")
    wid = c * NS + s
    pltpu.sync_copy(zeros_hbm, zero_v)
    for k in range(RPT // ZR):
        pltpu.sync_copy(zero_v, acc_sh.at[pl.ds(s * RPT + k * ZR, ZR)])

    @pl.when(s == 0)
    def _():
        pltpu.sync_copy(zero_v.at[pl.ds(0, TAIL)],
                        acc_sh.at[pl.ds(NS * RPT, TAIL)])

    dbufs = (dst0_v, dst1_v, dst2_v)
    for j in range(NSLOT):
        pltpu.sync_copy(src_hbm.at[wid, j], sidx_v.at[j])
    plsc.subcore_barrier()

    for j in range(2):
        pltpu.async_copy(dst_hbm.at[wid, j], dbufs[j], dsems.at[j])
        pltpu.async_copy(hs_hbm.at[sidx_v.at[j]], rows_v.at[j], gsems.at[j])

    def scat_wait(j):
        pltpu.make_async_copy(rows_v.at[j], acc_sh.at[dbufs[j]],
                              ssems.at[j]).wait()

    def visit(ci, j, wait_prev_scatter=True):
        jr = (j + 2) % NSLOT
        pltpu.make_async_copy(hs_hbm.at[sidx_v.at[j]], rows_v.at[j],
                              gsems.at[j]).wait()

        @pl.when(ci + NSLOT < NCH)
        def _():
            pltpu.async_copy(src_hbm.at[wid, ci + NSLOT], sidx_v.at[j],
                             isems.at[j])

        pltpu.make_async_copy(dst_hbm.at[wid, 0], dbufs[j],
                              dsems.at[j]).wait()
        pltpu.async_copy(rows_v.at[j], acc_sh.at[dbufs[j]], ssems.at[j],
                         add=True)

        @pl.when(ci + 2 < NCH)
        def _():
            if wait_prev_scatter:
                scat_wait(jr)
                pltpu.make_async_copy(src_hbm.at[wid, 0], sidx_v.at[jr],
                                      isems.at[jr]).wait()
            pltpu.async_copy(hs_hbm.at[sidx_v.at[jr]], rows_v.at[jr],
                             gsems.at[jr])
            pltpu.async_copy(dst_hbm.at[wid, ci + 2], dbufs[jr],
                             dsems.at[jr])

    visit(0, 0, wait_prev_scatter=False)
    visit(1, 1)
    visit(2, 2)

    def body(g, carry):
        visit(3 * g, 0)
        visit(3 * g + 1, 1)
        visit(3 * g + 2, 2)
        return carry

    lax.fori_loop(1, (NCH - 3) // 3 + 1, body, 0)
    visit(NCH - 2, (NCH - 2) % NSLOT)
    visit(NCH - 1, (NCH - 1) % NSLOT)
    for j in range(NSLOT):
        scat_wait(j)
    plsc.subcore_barrier()
    pltpu.sync_copy(acc_sh.at[pl.ds(s * RPT, RPT)],
                    out_hbm.at[c, pl.ds(s * RPT, RPT)])

    @pl.when(s == 0)
    def _():
        pltpu.sync_copy(acc_sh.at[pl.ds(NS * RPT, TAIL)],
                        out_hbm.at[c, pl.ds(NS * RPT, TAIL)])


def _row_spec(w):
    return pl.BlockSpec((BLK, w), lambda i: (i, 0))


def _row_spec2(w):
    return pl.BlockSpec((BLK, w), lambda i: (i % NBLK, 0))


def _pair_spec():
    return pl.BlockSpec((2, BLK, D), lambda i: (0, i, 0))


def _pair_spec2():
    return pl.BlockSpec((2, BLK, D), lambda i: (0, i % NBLK, 0))


def _full_spec(shape):
    nd = len(shape)
    return pl.BlockSpec(shape, lambda i: (0,) * nd)


def _dinv(dp):
    deg = 1.0 + dp[0, :, 0:1] + dp[1, :, 0:1]
    return lax.rsqrt(deg)


def _kan_body(x_ref, w1t_ref, b1_ref, w2t_ref, b2_ref, h_ref):
    h = jnp.dot(x_ref[...], w1t_ref[...], preferred_element_type=jnp.float32)
    h = jax.nn.gelu(h + b1_ref[...])
    h = jnp.dot(h, w2t_ref[...], preferred_element_type=jnp.float32) + b2_ref[...]
    h_ref[...] = h


def _tc_kan(x, w1t, b1, w2t, b2):
    return pl.pallas_call(
        _kan_body,
        grid=(NBLK,),
        in_specs=[_row_spec(D),
                  _full_spec((D, D)), _full_spec((1, D)),
                  _full_spec((D, D)), _full_spec((1, D))],
        out_specs=_row_spec(D),
        out_shape=jax.ShapeDtypeStruct((N, D), jnp.float32),
    )(x, w1t, b1, w2t, b2)


def _scale_body(h_ref, dp_ref, hs_ref):
    hs_ref[...] = h_ref[...] * _dinv(dp_ref[...])


def _tc_scale(h, dp):
    return pl.pallas_call(
        _scale_body,
        grid=(NBLK,),
        in_specs=[_row_spec(D), _pair_spec()],
        out_specs=_row_spec(D),
        out_shape=jax.ShapeDtypeStruct((N, D), jnp.float32),
    )(h, dp)


def _accum_stats(v, st_s):
    s1 = jnp.sum(v, axis=0, keepdims=True)
    s2 = jnp.sum(v * v, axis=0, keepdims=True)
    st = jnp.concatenate([s1, s2, jnp.zeros((6, D), v.dtype)], axis=0)

    @pl.when(pl.program_id(0) == 0)
    def _():
        st_s[...] = st

    @pl.when(pl.program_id(0) > 0)
    def _():
        st_s[...] = st_s[...] + st


def _bn(v, st, gamma, beta):
    mu = st[0:1, :] * (1.0 / N)
    var = st[1:2, :] * (1.0 / N) - mu * mu
    return gamma * (v - mu) * lax.rsqrt(var + 1e-5) + beta


def _comb_phase1(ap_ref, hs_ref, dp_ref, b_ref, v_s, st_s):
    dinv = _dinv(dp_ref[...])
    v = dinv * (ap_ref[0] + ap_ref[1] + hs_ref[...]) + b_ref[...]
    i = pl.program_id(0)
    v_s[pl.ds(i * BLK, BLK), :] = v
    _accum_stats(v, st_s)


def _bnkan_body(ap_ref, hs_ref, dp_ref, b_ref, g_ref, be_ref,
                w1t_ref, b1_ref, w2t_ref, b2_ref, h_ref, hs1_ref, v_s, st_s):
    i = pl.program_id(0)

    @pl.when(i < NBLK)
    def _():
        _comb_phase1(ap_ref, hs_ref, dp_ref, b_ref, v_s, st_s)

    @pl.when(i >= NBLK)
    def _():
        dinv = _dinv(dp_ref[...])
        v = v_s[pl.ds((i - NBLK) * BLK, BLK), :]
        h = _bn(v, st_s[...], g_ref[...], be_ref[...])
        h_ref[...] = h
        t = jnp.dot(h, w1t_ref[...], preferred_element_type=jnp.float32)
        t = jax.nn.gelu(t + b1_ref[...])
        t = jnp.dot(t, w2t_ref[...],
                    preferred_element_type=jnp.float32) + b2_ref[...]
        hs1_ref[...] = t * dinv


def _tc_comb_bn_kan(ap, hs, dp, bias, gamma, beta, w1t, b1, w2t, b2):
    return pl.pallas_call(
        _bnkan_body,
        grid=(2 * NBLK,),
        in_specs=[_pair_spec2(), _row_spec2(D), _pair_spec2(),
                  _full_spec((1, D)),
                  _full_spec((1, D)), _full_spec((1, D)),
                  _full_spec((D, D)), _full_spec((1, D)),
                  _full_spec((D, D)), _full_spec((1, D))],
        out_specs=[_row_spec2(D), _row_spec2(D)],
        out_shape=[jax.ShapeDtypeStruct((N, D), jnp.float32),
                   jax.ShapeDtypeStruct((N, D), jnp.float32)],
        scratch_shapes=[pltpu.VMEM((N, D), jnp.float32),
                        pltpu.VMEM((8, D), jnp.float32)],
    )(ap, hs, dp, bias, gamma, beta, w1t, b1, w2t, b2)


_G = [i * 0.5 - 2.5 for i in range(11)]


def _spline_head(z, bwt, swt):
    out = jnp.dot(jax.nn.silu(z), bwt, preferred_element_type=jnp.float32)
    bas = [((z >= _G[k]) & (z < _G[k + 1])).astype(z.dtype)
           for k in range(10)]
    for p in range(1, 4):
        bas = [(z - _G[k]) / (_G[k + p] - _G[k]) * bas[k]
               + (_G[k + p + 1] - z) / (_G[k + p + 1] - _G[k + 1])
               * bas[k + 1]
               for k in range(10 - p)]
    for j in range(7):
        out = out + jnp.dot(bas[j], swt[j],
                            preferred_element_type=jnp.float32)
    return out


def _head_pre_body(x_ref, h1_ref, bwt_ref, swt_ref, out_ref):
    z = jnp.concatenate([x_ref[...], h1_ref[...]], axis=1)
    out_ref[...] = _spline_head(z, bwt_ref[...], swt_ref[...])


def _tc_head_pre(x, h1, bwt_xh, swt_xh):
    return pl.pallas_call(
        _head_pre_body,
        grid=(NBLK,),
        in_specs=[_row_spec(D), _row_spec(D),
                  _full_spec((2 * D, C_OUT)), _full_spec((7, 2 * D, C_OUT))],
        out_specs=_row_spec(C_OUT),
        out_shape=jax.ShapeDtypeStruct((N, C_OUT), jnp.float32),
    )(x, h1, bwt_xh, swt_xh)


def _head_body(ap_ref, hs_ref, dp_ref, b_ref, g_ref, be_ref,
               pre_ref, bwt_ref, swt_ref, out_ref, v_s, st_s):
    i = pl.program_id(0)

    @pl.when(i < NBLK)
    def _():
        _comb_phase1(ap_ref, hs_ref, dp_ref, b_ref, v_s, st_s)

    @pl.when(i >= NBLK)
    def _():
        v2 = v_s[pl.ds((i - NBLK) * BLK, BLK), :]
        h2 = _bn(v2, st_s[...], g_ref[...], be_ref[...])
        out_ref[...] = pre_ref[...] + _spline_head(h2, bwt_ref[...],
                                                   swt_ref[...])


def _tc_comb_head(ap, hs, dp, bias, gamma, beta, pre, bwt_h2, swt_h2):
    return pl.pallas_call(
        _head_body,
        grid=(2 * NBLK,),
        in_specs=[_pair_spec2(), _row_spec2(D), _pair_spec2(),
                  _full_spec((1, D)),
                  _full_spec((1, D)), _full_spec((1, D)),
                  _row_spec2(C_OUT),
                  _full_spec((D, C_OUT)), _full_spec((7, D, C_OUT))],
        out_specs=_row_spec2(C_OUT),
        out_shape=jax.ShapeDtypeStruct((N, C_OUT), jnp.float32),
        scratch_shapes=[pltpu.VMEM((N, D), jnp.float32),
                        pltpu.VMEM((8, D), jnp.float32)],
    )(ap, hs, dp, bias, gamma, beta, pre, bwt_h2, swt_h2)


def kernel(x, edge_index, fc1_w0, fc1_b0, fc2_w0, fc2_b0, conv_b0, gamma0,
           beta0, fc1_w1, fc1_b1, fc2_w1, fc2_b1, conv_b1, gamma1, beta1,
           base_w, spline_w):
    src = edge_index[0].reshape(NW, NCH, CH)
    dst = edge_index[1].reshape(NW, NCH, CH)
    zeros_d = jnp.zeros((ZR, D), jnp.float32)
    ones_nd = jnp.ones((N, D), jnp.float32)
    degp = _sc_aggregate(src, dst, ones_nd, zeros_d)

    h0 = _tc_kan(x, fc1_w0.T, fc1_b0[None], fc2_w0.T, fc2_b0[None])
    hs0 = _tc_scale(h0, degp)
    acc0 = _sc_aggregate(src, dst, hs0, zeros_d)
    h1, hs1 = _tc_comb_bn_kan(acc0, hs0, degp, conv_b0[None],
                              gamma0[None], beta0[None], fc1_w1.T,
                              fc1_b1[None], fc2_w1.T, fc2_b1[None])

    swt = jnp.transpose(spline_w, (2, 1, 0))
    bwt = base_w.T
    acc1 = _sc_aggregate(src, dst, hs1, zeros_d)
    pre = _tc_head_pre(x, h1, bwt[:2 * D], swt[:, :2 * D, :])
    return _tc_comb_head(acc1, hs1, degp, conv_b1[None],
                         gamma1[None], beta1[None], pre,
                         bwt[2 * D:], swt[:, 2 * D:, :])
